# baseline (speedup 1.0000x reference)
import functools

import jax
import jax.numpy as jnp
from jax import lax
from jax.experimental import pallas as pl
from jax.experimental.pallas import tpu as pltpu
from jax.experimental.pallas import tpu_sc as plsc

NUM_FIELDS = 26
VOCAB = 100000
EMBED_DIM = 16
BATCH = 16384

NUM_WORKERS = 32
BPW = BATCH // NUM_WORKERS
ROWS_PW = BPW * NUM_FIELDS
IDX_ROWS = ROWS_PW // 128
CHUNK = 64
CROWS = CHUNK * NUM_FIELDS
CSUB = CROWS // 128
NCHUNK = BPW // CHUNK
GROUPS = CHUNK // 16


def _make_sc_kernel():
    mesh = plsc.VectorSubcoreMesh(core_axis_name="c", subcore_axis_name="s")

    @functools.partial(
        pl.kernel,
        mesh=mesh,
        compiler_params=pltpu.CompilerParams(
            needs_layout_passes=False, use_tc_tiling_on_sc=False),
        out_type=jax.ShapeDtypeStruct((BATCH,), jnp.float32),
        scratch_types=[
            pltpu.VMEM((IDX_ROWS, 128), jnp.int32),
            pltpu.VMEM((CROWS, EMBED_DIM), jnp.float32),
            pltpu.VMEM((CROWS + 32,), jnp.float32),
            pltpu.VMEM((256,), jnp.float32),
            pltpu.VMEM((BPW,), jnp.float32),
            pltpu.VMEM((16,), jnp.float32),
            pltpu.SemaphoreType.DMA,
            pltpu.SemaphoreType.DMA,
        ],
    )
    def fm_sc(xg_hbm, intf_hbm, linf_hbm, bias_hbm, out_hbm,
              x_v, rows_v, lin_v, pbuf, out_v, bias_v, sem_i, sem_l):
        wid = lax.axis_index("s") * 2 + lax.axis_index("c")
        pltpu.sync_copy(xg_hbm.at[pl.ds(wid * IDX_ROWS, IDX_ROWS)], x_v)
        pltpu.sync_copy(bias_hbm, bias_v)

        lanes = lax.iota(jnp.int32, 16)
        lin_mask = jnp.where(lanes < (NUM_FIELDS - 16), 1.0, 0.0)

        def chunk_body(c, _):
            copies = []
            for j in range(CSUB):
                idx_row = x_v.at[c * CSUB + j]
                copies.append(pltpu.async_copy(
                    intf_hbm.at[idx_row],
                    rows_v.at[pl.ds(j * 128, 128)], sem_i))
                copies.append(pltpu.async_copy(
                    linf_hbm.at[idx_row],
                    lin_v.at[pl.ds(j * 128, 128)], sem_l))
            for cp in copies:
                cp.wait()

            def group_body(g, _):
                res = jnp.zeros((16,), jnp.float32)
                for i in range(16):
                    item = g * 16 + i
                    base = item * NUM_FIELDS
                    v = rows_v[base]
                    s = v
                    q = v * v
                    for f in range(1, NUM_FIELDS):
                        v = rows_v[base + f]
                        s = s + v
                        q = q + v * v
                    l1 = lin_v[pl.ds(base, 16)]
                    l2 = lin_v[pl.ds(base + 16, 16)]
                    p = 0.5 * (s * s - q) + l1 + l2 * lin_mask
                    res = jnp.where(lanes == i, jnp.sum(p), res)
                out_v[pl.ds(c * CHUNK + g * 16, 16)] = res + bias_v[...]
                return _

            lax.fori_loop(0, GROUPS, group_body, None)
            return _

        lax.fori_loop(0, NCHUNK, chunk_body, None)
        pltpu.sync_copy(out_v, out_hbm.at[pl.ds(wid * BPW, BPW)])

    return fm_sc


_FM_SC = _make_sc_kernel()


@jax.jit
def kernel(X, int_tables, lin_tables, bias):
    offs = jnp.arange(NUM_FIELDS, dtype=jnp.int32) * VOCAB
    xg = (X.astype(jnp.int32) + offs[None, :]).reshape(
        NUM_WORKERS * IDX_ROWS, 128)
    intf = int_tables.reshape(NUM_FIELDS * VOCAB, EMBED_DIM)
    linf = lin_tables.reshape(NUM_FIELDS * VOCAB)
    bias16 = jnp.broadcast_to(bias.reshape(()), (16,)).astype(jnp.float32)
    out = _FM_SC(xg, intf, linf, bias16)
    return out.reshape(BATCH, 1)


# speedup vs baseline: 1.2237x; 1.2237x over previous
import functools

import jax
import jax.numpy as jnp
from jax import lax
from jax.experimental import pallas as pl
from jax.experimental.pallas import tpu as pltpu
from jax.experimental.pallas import tpu_sc as plsc

NUM_FIELDS = 26
VOCAB = 100000
VOCAB_PAD = 100096
NTILE = VOCAB_PAD // 128
EMBED_DIM = 16
BATCH = 16384

NUM_WORKERS = 32
UNITS = NUM_FIELDS * NTILE
UNITS_PW = -(-UNITS // NUM_WORKERS)

BPW = BATCH // NUM_WORKERS
ROWS_PW = BPW * NUM_FIELDS
IDX_ROWS = ROWS_PW // 128
CHUNK = 64
CROWS = CHUNK * NUM_FIELDS
CSUB = CROWS // 128
NCHUNK = BPW // CHUNK
GROUPS = CHUNK // 16

L_ROWS = NUM_FIELDS * VOCAB_PAD


def _make_transpose_kernel():
    mesh = plsc.VectorSubcoreMesh(core_axis_name="c", subcore_axis_name="s")

    @functools.partial(
        pl.kernel,
        mesh=mesh,
        compiler_params=pltpu.CompilerParams(
            needs_layout_passes=False, use_tc_tiling_on_sc=True),
        out_type=jax.ShapeDtypeStruct((L_ROWS * EMBED_DIM,), jnp.float32),
        scratch_types=[
            pltpu.VMEM((8, 128), jnp.float32),
            pltpu.VMEM((8, 128), jnp.float32),
            pltpu.VMEM((2048,), jnp.float32),
            pltpu.SemaphoreType.DMA,
        ],
    )
    def detile(a_hbm, l_hbm, b0, b1, obuf, sem):
        wid = lax.axis_index("s") * 2 + lax.axis_index("c")
        scat = lax.iota(jnp.int32, 16) * 16

        def unit_body(k, _):
            u = k * NUM_WORKERS + wid

            @pl.when(u < UNITS)
            def _():
                f = u // NTILE
                j = u % NTILE
                cp0 = pltpu.async_copy(
                    a_hbm.at[pl.ds(16 * f, 8), pl.ds(128 * j, 128)], b0, sem)
                cp1 = pltpu.async_copy(
                    a_hbm.at[pl.ds(16 * f + 8, 8), pl.ds(128 * j, 128)],
                    b1, sem)
                cp0.wait()
                cp1.wait()
                for d in range(8):
                    for cw in range(8):
                        v0 = b0[d, pl.ds(cw * 16, 16)]
                        plsc.store_scatter(obuf, [scat + (cw * 256 + d)], v0)
                        v1 = b1[d, pl.ds(cw * 16, 16)]
                        plsc.store_scatter(
                            obuf, [scat + (cw * 256 + d + 8)], v1)
                base = (f * VOCAB_PAD + 128 * j) * EMBED_DIM
                pltpu.sync_copy(obuf, l_hbm.at[pl.ds(base, 2048)])
            return _

        lax.fori_loop(0, UNITS_PW, unit_body, None)

    return detile


def _make_fm_kernel():
    mesh = plsc.VectorSubcoreMesh(core_axis_name="c", subcore_axis_name="s")

    @functools.partial(
        pl.kernel,
        mesh=mesh,
        compiler_params=pltpu.CompilerParams(
            needs_layout_passes=False, use_tc_tiling_on_sc=False),
        out_type=jax.ShapeDtypeStruct((BATCH,), jnp.float32),
        scratch_types=[
            pltpu.VMEM((IDX_ROWS, 128), jnp.int32),
            pltpu.VMEM((IDX_ROWS, 128), jnp.int32),
            pltpu.VMEM((CROWS, EMBED_DIM), jnp.float32),
            pltpu.VMEM((CROWS + 32,), jnp.float32),
            pltpu.VMEM((BPW,), jnp.float32),
            pltpu.VMEM((16,), jnp.float32),
            pltpu.SemaphoreType.DMA,
            pltpu.SemaphoreType.DMA,
        ],
    )
    def fm_sc(xg_hbm, xl_hbm, intf_hbm, linf_hbm, bias_hbm, out_hbm,
              x_v, xl_v, rows_v, lin_v, out_v, bias_v, sem_i, sem_l):
        wid = lax.axis_index("s") * 2 + lax.axis_index("c")
        pltpu.sync_copy(xg_hbm.at[pl.ds(wid * IDX_ROWS, IDX_ROWS)], x_v)
        pltpu.sync_copy(xl_hbm.at[pl.ds(wid * IDX_ROWS, IDX_ROWS)], xl_v)
        pltpu.sync_copy(bias_hbm, bias_v)

        lanes = lax.iota(jnp.int32, 16)
        lin_mask = jnp.where(lanes < (NUM_FIELDS - 16), 1.0, 0.0)

        def chunk_body(c, _):
            copies = []
            for j in range(CSUB):
                copies.append(pltpu.async_copy(
                    intf_hbm.at[x_v.at[c * CSUB + j]],
                    rows_v.at[pl.ds(j * 128, 128)], sem_i))
                copies.append(pltpu.async_copy(
                    linf_hbm.at[xl_v.at[c * CSUB + j]],
                    lin_v.at[pl.ds(j * 128, 128)], sem_l))
            for cp in copies:
                cp.wait()

            def group_body(g, _):
                res = jnp.zeros((16,), jnp.float32)
                for i in range(16):
                    item = g * 16 + i
                    base = item * NUM_FIELDS
                    v = rows_v[base]
                    s = v
                    q = v * v
                    for f in range(1, NUM_FIELDS):
                        v = rows_v[base + f]
                        s = s + v
                        q = q + v * v
                    l1 = lin_v[pl.ds(base, 16)]
                    l2 = lin_v[pl.ds(base + 16, 16)]
                    p = 0.5 * (s * s - q) + l1 + l2 * lin_mask
                    res = jnp.where(lanes == i, jnp.sum(p), res)
                out_v[pl.ds(c * CHUNK + g * 16, 16)] = res + bias_v[...]
                return _

            lax.fori_loop(0, GROUPS, group_body, None)
            return _

        lax.fori_loop(0, NCHUNK, chunk_body, None)
        pltpu.sync_copy(out_v, out_hbm.at[pl.ds(wid * BPW, BPW)])

    return fm_sc


_DETILE = _make_transpose_kernel()
_FM_SC = _make_fm_kernel()


@jax.jit
def kernel(X, int_tables, lin_tables, bias):
    a_view = jnp.transpose(int_tables, (0, 2, 1)).reshape(
        NUM_FIELDS * EMBED_DIM, VOCAB)
    l_flat = _DETILE(a_view)
    intf = l_flat.reshape(L_ROWS, EMBED_DIM)

    offs = jnp.arange(NUM_FIELDS, dtype=jnp.int32) * VOCAB_PAD
    xg = (X.astype(jnp.int32) + offs[None, :]).reshape(
        NUM_WORKERS * IDX_ROWS, 128)
    loffs = jnp.arange(NUM_FIELDS, dtype=jnp.int32) * VOCAB
    xl = (X.astype(jnp.int32) + loffs[None, :]).reshape(
        NUM_WORKERS * IDX_ROWS, 128)
    linf = lin_tables.reshape(NUM_FIELDS * VOCAB)
    bias16 = jnp.broadcast_to(bias.reshape(()), (16,)).astype(jnp.float32)
    out = _FM_SC(xg, xl, intf, linf, bias16)
    return out.reshape(BATCH, 1)


# speedup vs baseline: 2.2353x; 1.8267x over previous
import functools

import jax
import jax.numpy as jnp
from jax import lax
from jax.experimental import pallas as pl
from jax.experimental.pallas import tpu as pltpu
from jax.experimental.pallas import tpu_sc as plsc

NUM_FIELDS = 26
VOCAB = 100000
VOCAB_PAD = 100096
NTILE = VOCAB_PAD // 128
EMBED_DIM = 16
BATCH = 16384

NUM_WORKERS = 32
UNITS = NUM_FIELDS * NTILE
UNITS_PW = -(-UNITS // NUM_WORKERS)

BPW = BATCH // NUM_WORKERS
ROWS_PW = BPW * NUM_FIELDS
IDX_ROWS = ROWS_PW // 128
CHUNK = 64
CROWS = CHUNK * NUM_FIELDS
CSUB = CROWS // 128
NCHUNK = BPW // CHUNK
GROUPS = CHUNK // 16

L_ROWS = NUM_FIELDS * VOCAB_PAD


def _make_transpose_kernel():
    mesh = plsc.VectorSubcoreMesh(core_axis_name="c", subcore_axis_name="s")

    @functools.partial(
        pl.kernel,
        mesh=mesh,
        compiler_params=pltpu.CompilerParams(
            needs_layout_passes=False, use_tc_tiling_on_sc=True),
        out_type=jax.ShapeDtypeStruct((L_ROWS * EMBED_DIM,), jnp.float32),
        scratch_types=[
            pltpu.VMEM((8, 128), jnp.float32),
            pltpu.VMEM((8, 128), jnp.float32),
            pltpu.VMEM((8, 128), jnp.float32),
            pltpu.VMEM((8, 128), jnp.float32),
            pltpu.VMEM((2048,), jnp.float32),
            pltpu.VMEM((2048,), jnp.float32),
            pltpu.SemaphoreType.DMA,
            pltpu.SemaphoreType.DMA,
        ],
    )
    def detile(a_hbm, l_hbm, b0a, b0b, b1a, b1b, oa, ob, sem_in, sem_out):
        wid = lax.axis_index("s") * 2 + lax.axis_index("c")
        scat = lax.iota(jnp.int32, 16) * 16
        bufs = ((b0a, b1a, oa), (b0b, b1b, ob))

        def fj(k):
            u = jnp.minimum(k * NUM_WORKERS + wid, UNITS - 1)
            return u // NTILE, u % NTILE

        def issue_in(k, par):
            f, j = fj(k)
            b0, b1, _ = bufs[par]
            pltpu.async_copy(
                a_hbm.at[pl.ds(16 * f, 8), pl.ds(128 * j, 128)], b0, sem_in)
            pltpu.async_copy(
                a_hbm.at[pl.ds(16 * f + 8, 8), pl.ds(128 * j, 128)],
                b1, sem_in)

        def wait_in(par):
            b0, b1, _ = bufs[par]
            pltpu.make_async_copy(
                a_hbm.at[pl.ds(0, 8), pl.ds(0, 128)], b0, sem_in).wait()
            pltpu.make_async_copy(
                a_hbm.at[pl.ds(0, 8), pl.ds(0, 128)], b1, sem_in).wait()

        def wait_out(par):
            pltpu.make_async_copy(
                l_hbm.at[pl.ds(0, 2048)], bufs[par][2], sem_out).wait()

        issue_in(0, 0)
        issue_in(1, 1)

        def pair_body(kk, _):
            for par in range(2):
                k = kk * 2 + par
                b0, b1, obuf = bufs[par]
                wait_in(par)

                @pl.when(kk > 0)
                def _():
                    wait_out(par)

                for d in range(8):
                    for cw in range(8):
                        v0 = b0[d, pl.ds(cw * 16, 16)]
                        plsc.store_scatter(
                            obuf, [scat + (cw * 256 + d)], v0)
                        v1 = b1[d, pl.ds(cw * 16, 16)]
                        plsc.store_scatter(
                            obuf, [scat + (cw * 256 + d + 8)], v1)
                f, j = fj(k)
                base = (f * VOCAB_PAD + 128 * j) * EMBED_DIM
                pltpu.async_copy(
                    obuf, l_hbm.at[pl.ds(base, 2048)], sem_out)

                @pl.when(k + 2 < UNITS_PW)
                def _():
                    issue_in(k + 2, par)
            return _

        lax.fori_loop(0, UNITS_PW // 2, pair_body, None)
        wait_out(0)
        wait_out(1)

    return detile


def _make_fm_kernel():
    mesh = plsc.VectorSubcoreMesh(core_axis_name="c", subcore_axis_name="s")

    @functools.partial(
        pl.kernel,
        mesh=mesh,
        compiler_params=pltpu.CompilerParams(
            needs_layout_passes=False, use_tc_tiling_on_sc=False),
        out_type=jax.ShapeDtypeStruct((BATCH,), jnp.float32),
        scratch_types=[
            pltpu.VMEM((IDX_ROWS, 128), jnp.int32),
            pltpu.VMEM((IDX_ROWS, 128), jnp.int32),
            pltpu.VMEM((CROWS, EMBED_DIM), jnp.float32),
            pltpu.VMEM((CROWS + 32,), jnp.float32),
            pltpu.VMEM((BPW,), jnp.float32),
            pltpu.VMEM((16,), jnp.float32),
            pltpu.SemaphoreType.DMA,
            pltpu.SemaphoreType.DMA,
        ],
    )
    def fm_sc(xg_hbm, xl_hbm, intf_hbm, linf_hbm, bias_hbm, out_hbm,
              x_v, xl_v, rows_v, lin_v, out_v, bias_v, sem_i, sem_l):
        wid = lax.axis_index("s") * 2 + lax.axis_index("c")
        pltpu.sync_copy(xg_hbm.at[pl.ds(wid * IDX_ROWS, IDX_ROWS)], x_v)
        pltpu.sync_copy(xl_hbm.at[pl.ds(wid * IDX_ROWS, IDX_ROWS)], xl_v)
        pltpu.sync_copy(bias_hbm, bias_v)

        lanes = lax.iota(jnp.int32, 16)
        lin_mask = jnp.where(lanes < (NUM_FIELDS - 16), 1.0, 0.0)

        def chunk_body(c, _):
            copies = []
            for j in range(CSUB):
                copies.append(pltpu.async_copy(
                    intf_hbm.at[x_v.at[c * CSUB + j]],
                    rows_v.at[pl.ds(j * 128, 128)], sem_i))
                copies.append(pltpu.async_copy(
                    linf_hbm.at[xl_v.at[c * CSUB + j]],
                    lin_v.at[pl.ds(j * 128, 128)], sem_l))
            for cp in copies:
                cp.wait()

            def group_body(g, _):
                res = jnp.zeros((16,), jnp.float32)
                for i in range(16):
                    item = g * 16 + i
                    base = item * NUM_FIELDS
                    v = rows_v[base]
                    s = v
                    q = v * v
                    for f in range(1, NUM_FIELDS):
                        v = rows_v[base + f]
                        s = s + v
                        q = q + v * v
                    l1 = lin_v[pl.ds(base, 16)]
                    l2 = lin_v[pl.ds(base + 16, 16)]
                    p = 0.5 * (s * s - q) + l1 + l2 * lin_mask
                    res = jnp.where(lanes == i, jnp.sum(p), res)
                out_v[pl.ds(c * CHUNK + g * 16, 16)] = res + bias_v[...]
                return _

            lax.fori_loop(0, GROUPS, group_body, None)
            return _

        lax.fori_loop(0, NCHUNK, chunk_body, None)
        pltpu.sync_copy(out_v, out_hbm.at[pl.ds(wid * BPW, BPW)])

    return fm_sc


_DETILE = _make_transpose_kernel()
_FM_SC = _make_fm_kernel()


@jax.jit
def kernel(X, int_tables, lin_tables, bias):
    a_view = jnp.transpose(int_tables, (0, 2, 1)).reshape(
        NUM_FIELDS * EMBED_DIM, VOCAB)
    l_flat = _DETILE(a_view)
    intf = l_flat.reshape(L_ROWS, EMBED_DIM)

    offs = jnp.arange(NUM_FIELDS, dtype=jnp.int32) * VOCAB_PAD
    xg = (X.astype(jnp.int32) + offs[None, :]).reshape(
        NUM_WORKERS * IDX_ROWS, 128)
    loffs = jnp.arange(NUM_FIELDS, dtype=jnp.int32) * VOCAB
    xl = (X.astype(jnp.int32) + loffs[None, :]).reshape(
        NUM_WORKERS * IDX_ROWS, 128)
    linf = lin_tables.reshape(NUM_FIELDS * VOCAB)
    bias16 = jnp.broadcast_to(bias.reshape(()), (16,)).astype(jnp.float32)
    out = _FM_SC(xg, xl, intf, linf, bias16)
    return out.reshape(BATCH, 1)


# speedup vs baseline: 2.9141x; 1.3036x over previous
import functools

import jax
import jax.numpy as jnp
from jax import lax
from jax.experimental import pallas as pl
from jax.experimental.pallas import tpu as pltpu
from jax.experimental.pallas import tpu_sc as plsc

NUM_FIELDS = 26
VOCAB = 100000
VOCAB_PAD = 100096
NTILE = VOCAB_PAD // 128
EMBED_DIM = 16
BATCH = 16384

NUM_WORKERS = 32
NTILE2 = NTILE // 2
UNITS = NUM_FIELDS * NTILE2
UNITS_PW = -(-UNITS // NUM_WORKERS)

BPW = BATCH // NUM_WORKERS
ROWS_PW = BPW * NUM_FIELDS
IDX_ROWS = ROWS_PW // 128
CHUNK = 64
CROWS = CHUNK * NUM_FIELDS
CSUB = CROWS // 128
NCHUNK = BPW // CHUNK
GROUPS = CHUNK // 16

L_ROWS = NUM_FIELDS * VOCAB_PAD


def _make_transpose_kernel():
    mesh = plsc.VectorSubcoreMesh(core_axis_name="c", subcore_axis_name="s")

    @functools.partial(
        pl.kernel,
        mesh=mesh,
        compiler_params=pltpu.CompilerParams(
            needs_layout_passes=False, use_tc_tiling_on_sc=True),
        out_type=jax.ShapeDtypeStruct((L_ROWS * EMBED_DIM,), jnp.float32),
        scratch_types=[
            pltpu.VMEM((8, 256), jnp.float32),
            pltpu.VMEM((8, 256), jnp.float32),
            pltpu.VMEM((8, 256), jnp.float32),
            pltpu.VMEM((8, 256), jnp.float32),
            pltpu.VMEM((4096,), jnp.float32),
            pltpu.VMEM((4096,), jnp.float32),
            pltpu.SemaphoreType.DMA,
            pltpu.SemaphoreType.DMA,
        ],
    )
    def detile(a_hbm, l_hbm, b0a, b0b, b1a, b1b, oa, ob, sem_in, sem_out):
        wid = lax.axis_index("s") * 2 + lax.axis_index("c")
        scat = lax.iota(jnp.int32, 16) * 16
        bufs = ((b0a, b1a, oa), (b0b, b1b, ob))

        def fj(k):
            u = jnp.minimum(k * NUM_WORKERS + wid, UNITS - 1)
            return u // NTILE2, u % NTILE2

        def issue_in(k, par):
            f, j = fj(k)
            b0, b1, _ = bufs[par]
            pltpu.async_copy(
                a_hbm.at[pl.ds(16 * f, 8), pl.ds(256 * j, 256)], b0, sem_in)
            pltpu.async_copy(
                a_hbm.at[pl.ds(16 * f + 8, 8), pl.ds(256 * j, 256)],
                b1, sem_in)

        def wait_in(par):
            b0, b1, _ = bufs[par]
            pltpu.make_async_copy(
                a_hbm.at[pl.ds(0, 8), pl.ds(0, 256)], b0, sem_in).wait()
            pltpu.make_async_copy(
                a_hbm.at[pl.ds(0, 8), pl.ds(0, 256)], b1, sem_in).wait()

        def wait_out(par):
            pltpu.make_async_copy(
                l_hbm.at[pl.ds(0, 4096)], bufs[par][2], sem_out).wait()

        issue_in(0, 0)
        issue_in(1, 1)

        def pair_body(kk, _):
            for par in range(2):
                k = kk * 2 + par
                b0, b1, obuf = bufs[par]
                wait_in(par)

                @pl.when(kk > 0)
                def _():
                    wait_out(par)

                for d in range(8):
                    idx_lo = scat + d
                    idx_hi = scat + (d + 8)
                    for cw in range(16):
                        tgt = obuf.at[pl.ds(cw * 256, 256)]
                        v0 = b0[d, pl.ds(cw * 16, 16)]
                        plsc.store_scatter(tgt, [idx_lo], v0)
                        v1 = b1[d, pl.ds(cw * 16, 16)]
                        plsc.store_scatter(tgt, [idx_hi], v1)
                f, j = fj(k)
                base = (f * VOCAB_PAD + 256 * j) * EMBED_DIM
                pltpu.async_copy(
                    obuf, l_hbm.at[pl.ds(base, 4096)], sem_out)

                @pl.when(k + 2 < UNITS_PW)
                def _():
                    issue_in(k + 2, par)
            return _

        lax.fori_loop(0, UNITS_PW // 2, pair_body, None)
        wait_out(0)
        wait_out(1)

    return detile


def _make_fm_kernel():
    mesh = plsc.VectorSubcoreMesh(core_axis_name="c", subcore_axis_name="s")

    @functools.partial(
        pl.kernel,
        mesh=mesh,
        compiler_params=pltpu.CompilerParams(
            needs_layout_passes=False, use_tc_tiling_on_sc=False),
        out_type=jax.ShapeDtypeStruct((BATCH,), jnp.float32),
        scratch_types=[
            pltpu.VMEM((IDX_ROWS, 128), jnp.int32),
            pltpu.VMEM((IDX_ROWS, 128), jnp.int32),
            pltpu.VMEM((CROWS, EMBED_DIM), jnp.float32),
            pltpu.VMEM((CROWS + 32,), jnp.float32),
            pltpu.VMEM((BPW,), jnp.float32),
            pltpu.VMEM((16,), jnp.float32),
            pltpu.SemaphoreType.DMA,
            pltpu.SemaphoreType.DMA,
        ],
    )
    def fm_sc(xg_hbm, xl_hbm, intf_hbm, linf_hbm, bias_hbm, out_hbm,
              x_v, xl_v, rows_v, lin_v, out_v, bias_v, sem_i, sem_l):
        wid = lax.axis_index("s") * 2 + lax.axis_index("c")
        pltpu.sync_copy(xg_hbm.at[pl.ds(wid * IDX_ROWS, IDX_ROWS)], x_v)
        pltpu.sync_copy(xl_hbm.at[pl.ds(wid * IDX_ROWS, IDX_ROWS)], xl_v)
        pltpu.sync_copy(bias_hbm, bias_v)

        lanes = lax.iota(jnp.int32, 16)
        lin_mask = jnp.where(lanes < (NUM_FIELDS - 16), 1.0, 0.0)

        def chunk_body(c, _):
            copies = []
            for j in range(CSUB):
                copies.append(pltpu.async_copy(
                    intf_hbm.at[x_v.at[c * CSUB + j]],
                    rows_v.at[pl.ds(j * 128, 128)], sem_i))
                copies.append(pltpu.async_copy(
                    linf_hbm.at[xl_v.at[c * CSUB + j]],
                    lin_v.at[pl.ds(j * 128, 128)], sem_l))
            for cp in copies:
                cp.wait()

            def group_body(g, _):
                res = jnp.zeros((16,), jnp.float32)
                for i in range(16):
                    item = g * 16 + i
                    base = item * NUM_FIELDS
                    v = rows_v[base]
                    s = v
                    q = v * v
                    for f in range(1, NUM_FIELDS):
                        v = rows_v[base + f]
                        s = s + v
                        q = q + v * v
                    l1 = lin_v[pl.ds(base, 16)]
                    l2 = lin_v[pl.ds(base + 16, 16)]
                    p = 0.5 * (s * s - q) + l1 + l2 * lin_mask
                    res = jnp.where(lanes == i, jnp.sum(p), res)
                out_v[pl.ds(c * CHUNK + g * 16, 16)] = res + bias_v[...]
                return _

            lax.fori_loop(0, GROUPS, group_body, None)
            return _

        lax.fori_loop(0, NCHUNK, chunk_body, None)
        pltpu.sync_copy(out_v, out_hbm.at[pl.ds(wid * BPW, BPW)])

    return fm_sc


_DETILE = _make_transpose_kernel()
_FM_SC = _make_fm_kernel()


@jax.jit
def kernel(X, int_tables, lin_tables, bias):
    a_view = jnp.transpose(int_tables, (0, 2, 1)).reshape(
        NUM_FIELDS * EMBED_DIM, VOCAB)
    l_flat = _DETILE(a_view)
    intf = l_flat.reshape(L_ROWS, EMBED_DIM)

    offs = jnp.arange(NUM_FIELDS, dtype=jnp.int32) * VOCAB_PAD
    xg = (X.astype(jnp.int32) + offs[None, :]).reshape(
        NUM_WORKERS * IDX_ROWS, 128)
    loffs = jnp.arange(NUM_FIELDS, dtype=jnp.int32) * VOCAB
    xl = (X.astype(jnp.int32) + loffs[None, :]).reshape(
        NUM_WORKERS * IDX_ROWS, 128)
    linf = lin_tables.reshape(NUM_FIELDS * VOCAB)
    bias16 = jnp.broadcast_to(bias.reshape(()), (16,)).astype(jnp.float32)
    out = _FM_SC(xg, xl, intf, linf, bias16)
    return out.reshape(BATCH, 1)


# speedup vs baseline: 2.9855x; 1.0245x over previous
import functools

import jax
import jax.numpy as jnp
from jax import lax
from jax.experimental import pallas as pl
from jax.experimental.pallas import tpu as pltpu
from jax.experimental.pallas import tpu_sc as plsc

NUM_FIELDS = 26
VOCAB = 100000
VOCAB_PAD = 100096
NTILE = VOCAB_PAD // 128
EMBED_DIM = 16
BATCH = 16384

NUM_WORKERS = 32
NTILE2 = NTILE // 2
UNITS = NUM_FIELDS * NTILE2
UNITS_PW = -(-UNITS // NUM_WORKERS)

BPW = BATCH // NUM_WORKERS
ROWS_PW = BPW * NUM_FIELDS
IDX_ROWS = ROWS_PW // 128
CHUNK = 64
CROWS = CHUNK * NUM_FIELDS
CSUB = CROWS // 128
NCHUNK = BPW // CHUNK
GROUPS = CHUNK // 16

L_ROWS = NUM_FIELDS * VOCAB_PAD


def _make_transpose_kernel():
    mesh = plsc.VectorSubcoreMesh(core_axis_name="c", subcore_axis_name="s")

    @functools.partial(
        pl.kernel,
        mesh=mesh,
        compiler_params=pltpu.CompilerParams(
            needs_layout_passes=False, use_tc_tiling_on_sc=True),
        out_type=jax.ShapeDtypeStruct((L_ROWS * EMBED_DIM,), jnp.float32),
        scratch_types=[
            pltpu.VMEM((8, 256), jnp.float32),
            pltpu.VMEM((8, 256), jnp.float32),
            pltpu.VMEM((8, 256), jnp.float32),
            pltpu.VMEM((8, 256), jnp.float32),
            pltpu.VMEM((4096,), jnp.float32),
            pltpu.VMEM((4096,), jnp.float32),
            pltpu.SemaphoreType.DMA,
            pltpu.SemaphoreType.DMA,
        ],
    )
    def detile(a_hbm, l_hbm, b0a, b0b, b1a, b1b, oa, ob, sem_in, sem_out):
        wid = lax.axis_index("s") * 2 + lax.axis_index("c")
        scat = lax.iota(jnp.int32, 16) * 16
        bufs = ((b0a, b1a, oa), (b0b, b1b, ob))

        def fj(k):
            u = jnp.minimum(k * NUM_WORKERS + wid, UNITS - 1)
            return u // NTILE2, u % NTILE2

        def issue_in(k, par):
            f, j = fj(k)
            b0, b1, _ = bufs[par]
            pltpu.async_copy(
                a_hbm.at[pl.ds(16 * f, 8), pl.ds(256 * j, 256)], b0, sem_in)
            pltpu.async_copy(
                a_hbm.at[pl.ds(16 * f + 8, 8), pl.ds(256 * j, 256)],
                b1, sem_in)

        def wait_in(par):
            b0, b1, _ = bufs[par]
            pltpu.make_async_copy(
                a_hbm.at[pl.ds(0, 8), pl.ds(0, 256)], b0, sem_in).wait()
            pltpu.make_async_copy(
                a_hbm.at[pl.ds(0, 8), pl.ds(0, 256)], b1, sem_in).wait()

        def wait_out(par):
            pltpu.make_async_copy(
                l_hbm.at[pl.ds(0, 4096)], bufs[par][2], sem_out).wait()

        issue_in(0, 0)
        issue_in(1, 1)

        def pair_body(kk, _):
            for par in range(2):
                k = kk * 2 + par
                b0, b1, obuf = bufs[par]
                wait_in(par)

                @pl.when(kk > 0)
                def _():
                    wait_out(par)

                for d in range(8):
                    idx_lo = scat + d
                    idx_hi = scat + (d + 8)
                    v0s = [b0[d, pl.ds(cw * 16, 16)] for cw in range(16)]
                    v1s = [b1[d, pl.ds(cw * 16, 16)] for cw in range(16)]
                    for cw in range(16):
                        tgt = obuf.at[pl.ds(cw * 256, 256)]
                        plsc.store_scatter(tgt, [idx_lo], v0s[cw])
                        plsc.store_scatter(tgt, [idx_hi], v1s[cw])
                f, j = fj(k)
                base = (f * VOCAB_PAD + 256 * j) * EMBED_DIM
                pltpu.async_copy(
                    obuf, l_hbm.at[pl.ds(base, 4096)], sem_out)

                @pl.when(k + 2 < UNITS_PW)
                def _():
                    issue_in(k + 2, par)
            return _

        lax.fori_loop(0, UNITS_PW // 2, pair_body, None)
        wait_out(0)
        wait_out(1)

    return detile


def _make_fm_kernel():
    mesh = plsc.VectorSubcoreMesh(core_axis_name="c", subcore_axis_name="s")

    @functools.partial(
        pl.kernel,
        mesh=mesh,
        compiler_params=pltpu.CompilerParams(
            needs_layout_passes=False, use_tc_tiling_on_sc=False),
        out_type=jax.ShapeDtypeStruct((BATCH,), jnp.float32),
        scratch_types=[
            pltpu.VMEM((IDX_ROWS, 128), jnp.int32),
            pltpu.VMEM((IDX_ROWS, 128), jnp.int32),
            pltpu.VMEM((CROWS, EMBED_DIM), jnp.float32),
            pltpu.VMEM((CROWS + 32,), jnp.float32),
            pltpu.VMEM((BPW,), jnp.float32),
            pltpu.VMEM((16,), jnp.float32),
            pltpu.SemaphoreType.DMA,
            pltpu.SemaphoreType.DMA,
        ],
    )
    def fm_sc(xg_hbm, xl_hbm, intf_hbm, linf_hbm, bias_hbm, out_hbm,
              x_v, xl_v, rows_v, lin_v, out_v, bias_v, sem_i, sem_l):
        wid = lax.axis_index("s") * 2 + lax.axis_index("c")
        pltpu.sync_copy(xg_hbm.at[pl.ds(wid * IDX_ROWS, IDX_ROWS)], x_v)
        pltpu.sync_copy(xl_hbm.at[pl.ds(wid * IDX_ROWS, IDX_ROWS)], xl_v)
        pltpu.sync_copy(bias_hbm, bias_v)

        lanes = lax.iota(jnp.int32, 16)
        lin_mask = jnp.where(lanes < (NUM_FIELDS - 16), 1.0, 0.0)

        def chunk_body(c, _):
            copies = []
            for j in range(CSUB):
                copies.append(pltpu.async_copy(
                    intf_hbm.at[x_v.at[c * CSUB + j]],
                    rows_v.at[pl.ds(j * 128, 128)], sem_i))
                copies.append(pltpu.async_copy(
                    linf_hbm.at[xl_v.at[c * CSUB + j]],
                    lin_v.at[pl.ds(j * 128, 128)], sem_l))
            for cp in copies:
                cp.wait()

            def group_body(g, _):
                res = jnp.zeros((16,), jnp.float32)
                for i in range(16):
                    item = g * 16 + i
                    base = item * NUM_FIELDS
                    v = rows_v[base]
                    s = v
                    q = v * v
                    for f in range(1, NUM_FIELDS):
                        v = rows_v[base + f]
                        s = s + v
                        q = q + v * v
                    l1 = lin_v[pl.ds(base, 16)]
                    l2 = lin_v[pl.ds(base + 16, 16)]
                    p = 0.5 * (s * s - q) + l1 + l2 * lin_mask
                    res = jnp.where(lanes == i, jnp.sum(p), res)
                out_v[pl.ds(c * CHUNK + g * 16, 16)] = res + bias_v[...]
                return _

            lax.fori_loop(0, GROUPS, group_body, None)
            return _

        lax.fori_loop(0, NCHUNK, chunk_body, None)
        pltpu.sync_copy(out_v, out_hbm.at[pl.ds(wid * BPW, BPW)])

    return fm_sc


_DETILE = _make_transpose_kernel()
_FM_SC = _make_fm_kernel()


@jax.jit
def kernel(X, int_tables, lin_tables, bias):
    a_view = jnp.transpose(int_tables, (0, 2, 1)).reshape(
        NUM_FIELDS * EMBED_DIM, VOCAB)
    l_flat = _DETILE(a_view)
    intf = l_flat.reshape(L_ROWS, EMBED_DIM)

    offs = jnp.arange(NUM_FIELDS, dtype=jnp.int32) * VOCAB_PAD
    xg = (X.astype(jnp.int32) + offs[None, :]).reshape(
        NUM_WORKERS * IDX_ROWS, 128)
    loffs = jnp.arange(NUM_FIELDS, dtype=jnp.int32) * VOCAB
    xl = (X.astype(jnp.int32) + loffs[None, :]).reshape(
        NUM_WORKERS * IDX_ROWS, 128)
    linf = lin_tables.reshape(NUM_FIELDS * VOCAB)
    bias16 = jnp.broadcast_to(bias.reshape(()), (16,)).astype(jnp.float32)
    out = _FM_SC(xg, xl, intf, linf, bias16)
    return out.reshape(BATCH, 1)
